# baseline (speedup 1.0000x reference)
import functools

import jax
import jax.numpy as jnp
from jax import lax
from jax.experimental import pallas as pl
from jax.experimental.pallas import tpu as pltpu
from jax.experimental.pallas import tpu_sc as plsc

N = 4194304
NC = 2
NS = 16
L = 16
NW = NC * NS
CHUNK = N // NW
CSUB = 32768
CNG = CHUNK // (2 * CSUB)
SUB = 16384
NG = CHUNK // (2 * SUB)

_mesh = plsc.VectorSubcoreMesh(
    core_axis_name="c", subcore_axis_name="s", num_cores=NC, num_subcores=NS
)


_TCROWS = 512
_TCGRID = N // 128 // _TCROWS


def _count_tc_body(seq_ref, out_ref, acc_ref):
    i = pl.program_id(0)

    @pl.when(i == 0)
    def _():
        acc_ref[...] = jnp.zeros((_TCROWS, 128), jnp.int32)

    acc_ref[...] += (seq_ref[...] == 1).astype(jnp.int32)

    @pl.when(i == _TCGRID - 1)
    def _():
        out_ref[...] = jnp.full((L,), jnp.sum(acc_ref[...]), jnp.int32)


def _scale_body(
    seq_hbm, part_hbm, p_hbm, out_hbm,
    pbuf, sbuf, in0, in1, r0a, r0b, r1a, r1b,
    isem0, isem1, os00, os01, os10, os11,
):
    wid = lax.axis_index("s") * NC + lax.axis_index("c")
    base = wid * CHUNK
    ibufs = (in0, in1)
    isems = (isem0, isem1)
    r0bufs = (r0a, r0b)
    r1bufs = (r1a, r1b)
    osems = ((os00, os01), (os10, os11))

    for b in range(2):
        pltpu.async_copy(
            seq_hbm.at[pl.ds(base + b * SUB, SUB)], ibufs[b], isems[b]
        )

    pltpu.sync_copy(part_hbm, pbuf)
    pltpu.sync_copy(p_hbm, sbuf)
    tot_f = pbuf[...].astype(jnp.float32)
    p_vec = sbuf[...]
    a_vec = p_vec / tot_f
    b_vec = (jnp.float32(1.0) - p_vec) / tot_f
    zero = jnp.zeros((L,), jnp.float32)

    def group(g, carry):
        for b in range(2):
            pltpu.make_async_copy(
                seq_hbm.at[pl.ds(base, SUB)], ibufs[b], isems[b]
            ).wait()

            @pl.when(g >= 1)
            def _(b=b):
                off0 = base
                pltpu.make_async_copy(
                    r0bufs[b], out_hbm.at[0, pl.ds(off0, SUB)], osems[0][b]
                ).wait()
                pltpu.make_async_copy(
                    r1bufs[b], out_hbm.at[1, pl.ds(off0, SUB)], osems[1][b]
                ).wait()

            @plsc.parallel_loop(0, SUB // L, step=4, unroll=4)
            def _(i, ibuf=ibufs[b], r0=r0bufs[b], r1=r1bufs[b]):
                for u in range(4):
                    off = (i + u) * L
                    x = ibuf[pl.ds(off, L)]
                    m = x == 1
                    r0[pl.ds(off, L)] = jnp.where(m, a_vec, zero)
                    r1[pl.ds(off, L)] = jnp.where(m, b_vec, zero)

            off = base + (2 * g + b) * SUB
            pltpu.async_copy(
                r0bufs[b], out_hbm.at[0, pl.ds(off, SUB)], osems[0][b]
            )
            pltpu.async_copy(
                r1bufs[b], out_hbm.at[1, pl.ds(off, SUB)], osems[1][b]
            )

            @pl.when(g + 1 < NG)
            def _(b=b):
                pltpu.async_copy(
                    seq_hbm.at[pl.ds(base + ((g + 1) * 2 + b) * SUB, SUB)],
                    ibufs[b],
                    isems[b],
                )

        return carry

    lax.fori_loop(0, NG, group, jnp.int32(0))
    for b in range(2):
        pltpu.make_async_copy(
            r0bufs[b], out_hbm.at[0, pl.ds(base, SUB)], osems[0][b]
        ).wait()
        pltpu.make_async_copy(
            r1bufs[b], out_hbm.at[1, pl.ds(base, SUB)], osems[1][b]
        ).wait()


_params = pltpu.CompilerParams(needs_layout_passes=False)

_count_call = pl.pallas_call(
    _count_tc_body,
    grid=(_TCGRID,),
    in_specs=[pl.BlockSpec((_TCROWS, 128), lambda i: (i, 0))],
    out_specs=pl.BlockSpec((L,), lambda i: (0,)),
    out_shape=jax.ShapeDtypeStruct((L,), jnp.int32),
    scratch_shapes=[pltpu.VMEM((_TCROWS, 128), jnp.int32)],
)

_scale_call = pl.kernel(
    _scale_body,
    out_type=jax.ShapeDtypeStruct((2, N), jnp.float32),
    mesh=_mesh,
    compiler_params=_params,
    scratch_types=[
        pltpu.VMEM((L,), jnp.int32),
        pltpu.VMEM((L,), jnp.float32),
        pltpu.VMEM((SUB,), jnp.int32),
        pltpu.VMEM((SUB,), jnp.int32),
        pltpu.VMEM((SUB,), jnp.float32),
        pltpu.VMEM((SUB,), jnp.float32),
        pltpu.VMEM((SUB,), jnp.float32),
        pltpu.VMEM((SUB,), jnp.float32),
        pltpu.SemaphoreType.DMA,
        pltpu.SemaphoreType.DMA,
        pltpu.SemaphoreType.DMA,
        pltpu.SemaphoreType.DMA,
        pltpu.SemaphoreType.DMA,
        pltpu.SemaphoreType.DMA,
    ],
)


def kernel(sequence, replication_prob, ber_short_patch_prob, ber_long_patch_prob):
    seq = sequence.astype(jnp.int32)
    p16 = jnp.broadcast_to(replication_prob.astype(jnp.float32), (L,))
    partials = _count_call(seq.reshape(N // 128, 128))
    return _scale_call(seq, partials, p16)


# speedup vs baseline: 1.4163x; 1.4163x over previous
import functools

import jax
import jax.numpy as jnp
from jax import lax
from jax.experimental import pallas as pl
from jax.experimental.pallas import tpu as pltpu
from jax.experimental.pallas import tpu_sc as plsc

N = 4194304
NC = 2
NS = 16
L = 16
NW = NC * NS
CHUNK = N // NW
CSUB = 32768
CNG = CHUNK // (2 * CSUB)
SUB = 16384
NG = CHUNK // (2 * SUB)

_mesh = plsc.VectorSubcoreMesh(
    core_axis_name="c", subcore_axis_name="s", num_cores=NC, num_subcores=NS
)


_TCROWS = 2048
_TCGRID = N // 128 // _TCROWS


def _count_tc_body(seq_ref, out_ref, acc_ref):
    i = pl.program_id(0)

    @pl.when(i == 0)
    def _():
        acc_ref[...] = jnp.zeros((_TCROWS, 128), jnp.int32)

    acc_ref[...] += (seq_ref[...] == 1).astype(jnp.int32)

    @pl.when(i == _TCGRID - 1)
    def _():
        out_ref[...] = jnp.full((L,), jnp.sum(acc_ref[...]), jnp.int32)


def _scale_body(
    seq_hbm, part_hbm, p_hbm, out_hbm,
    pbuf, sbuf, in0, in1, r0a, r0b, r1a, r1b,
    isem0, isem1, os00, os01, os10, os11,
):
    wid = lax.axis_index("s") * NC + lax.axis_index("c")
    base = wid * CHUNK
    ibufs = (in0, in1)
    isems = (isem0, isem1)
    r0bufs = (r0a, r0b)
    r1bufs = (r1a, r1b)
    osems = ((os00, os01), (os10, os11))

    for b in range(2):
        pltpu.async_copy(
            seq_hbm.at[pl.ds(base + b * SUB, SUB)], ibufs[b], isems[b]
        )

    pltpu.sync_copy(part_hbm, pbuf)
    pltpu.sync_copy(p_hbm, sbuf)
    tot_f = pbuf[...].astype(jnp.float32)
    p_vec = sbuf[...]
    a_vec = p_vec / tot_f
    b_vec = (jnp.float32(1.0) - p_vec) / tot_f
    zero = jnp.zeros((L,), jnp.float32)

    def group(g, carry):
        for b in range(2):
            pltpu.make_async_copy(
                seq_hbm.at[pl.ds(base, SUB)], ibufs[b], isems[b]
            ).wait()

            @pl.when(g >= 1)
            def _(b=b):
                off0 = base
                pltpu.make_async_copy(
                    r0bufs[b], out_hbm.at[0, pl.ds(off0, SUB)], osems[0][b]
                ).wait()
                pltpu.make_async_copy(
                    r1bufs[b], out_hbm.at[1, pl.ds(off0, SUB)], osems[1][b]
                ).wait()

            @plsc.parallel_loop(0, SUB // L, step=4, unroll=4)
            def _(i, ibuf=ibufs[b], r0=r0bufs[b], r1=r1bufs[b]):
                for u in range(4):
                    off = (i + u) * L
                    x = ibuf[pl.ds(off, L)]
                    m = x == 1
                    r0[pl.ds(off, L)] = jnp.where(m, a_vec, zero)
                    r1[pl.ds(off, L)] = jnp.where(m, b_vec, zero)

            off = base + (2 * g + b) * SUB
            pltpu.async_copy(
                r0bufs[b], out_hbm.at[0, pl.ds(off, SUB)], osems[0][b]
            )
            pltpu.async_copy(
                r1bufs[b], out_hbm.at[1, pl.ds(off, SUB)], osems[1][b]
            )

            @pl.when(g + 1 < NG)
            def _(b=b):
                pltpu.async_copy(
                    seq_hbm.at[pl.ds(base + ((g + 1) * 2 + b) * SUB, SUB)],
                    ibufs[b],
                    isems[b],
                )

        return carry

    lax.fori_loop(0, NG, group, jnp.int32(0))
    for b in range(2):
        pltpu.make_async_copy(
            r0bufs[b], out_hbm.at[0, pl.ds(base, SUB)], osems[0][b]
        ).wait()
        pltpu.make_async_copy(
            r1bufs[b], out_hbm.at[1, pl.ds(base, SUB)], osems[1][b]
        ).wait()


_params = pltpu.CompilerParams(needs_layout_passes=False)

_count_call = pl.pallas_call(
    _count_tc_body,
    grid=(_TCGRID,),
    in_specs=[pl.BlockSpec((_TCROWS, 128), lambda i: (i, 0))],
    out_specs=pl.BlockSpec((L,), lambda i: (0,)),
    out_shape=jax.ShapeDtypeStruct((L,), jnp.int32),
    scratch_shapes=[pltpu.VMEM((_TCROWS, 128), jnp.int32)],
)

_scale_call = pl.kernel(
    _scale_body,
    out_type=jax.ShapeDtypeStruct((2, N), jnp.float32),
    mesh=_mesh,
    compiler_params=_params,
    scratch_types=[
        pltpu.VMEM((L,), jnp.int32),
        pltpu.VMEM((L,), jnp.float32),
        pltpu.VMEM((SUB,), jnp.int32),
        pltpu.VMEM((SUB,), jnp.int32),
        pltpu.VMEM((SUB,), jnp.float32),
        pltpu.VMEM((SUB,), jnp.float32),
        pltpu.VMEM((SUB,), jnp.float32),
        pltpu.VMEM((SUB,), jnp.float32),
        pltpu.SemaphoreType.DMA,
        pltpu.SemaphoreType.DMA,
        pltpu.SemaphoreType.DMA,
        pltpu.SemaphoreType.DMA,
        pltpu.SemaphoreType.DMA,
        pltpu.SemaphoreType.DMA,
    ],
)


def kernel(sequence, replication_prob, ber_short_patch_prob, ber_long_patch_prob):
    seq = sequence.astype(jnp.int32)
    p16 = jnp.broadcast_to(replication_prob.astype(jnp.float32), (L,))
    partials = _count_call(seq.reshape(N // 128, 128))
    return _scale_call(seq, partials, p16)


# speedup vs baseline: 1.5257x; 1.0772x over previous
import functools

import jax
import jax.numpy as jnp
from jax import lax
from jax.experimental import pallas as pl
from jax.experimental.pallas import tpu as pltpu
from jax.experimental.pallas import tpu_sc as plsc

N = 4194304
NC = 2
NS = 16
L = 16
NW = NC * NS
CHUNK = N // NW
CSUB = 32768
CNG = CHUNK // (2 * CSUB)
SUB = 16384
NG = CHUNK // (2 * SUB)

_mesh = plsc.VectorSubcoreMesh(
    core_axis_name="c", subcore_axis_name="s", num_cores=NC, num_subcores=NS
)


_TCROWS = 4096
_TCGRID = N // 128 // _TCROWS


def _count_tc_body(seq_ref, out_ref, acc_ref):
    i = pl.program_id(0)

    @pl.when(i == 0)
    def _():
        acc_ref[...] = jnp.zeros((_TCROWS, 128), jnp.int32)

    acc_ref[...] += (seq_ref[...] == 1).astype(jnp.int32)

    @pl.when(i == _TCGRID - 1)
    def _():
        out_ref[...] = jnp.full((L,), jnp.sum(acc_ref[...]), jnp.int32)


def _scale_body(
    seq_hbm, part_hbm, p_hbm, out_hbm,
    pbuf, sbuf, in0, in1, r0a, r0b, r1a, r1b,
    isem0, isem1, os00, os01, os10, os11,
):
    wid = lax.axis_index("s") * NC + lax.axis_index("c")
    base = wid * CHUNK
    ibufs = (in0, in1)
    isems = (isem0, isem1)
    r0bufs = (r0a, r0b)
    r1bufs = (r1a, r1b)
    osems = ((os00, os01), (os10, os11))

    for b in range(2):
        pltpu.async_copy(
            seq_hbm.at[pl.ds(base + b * SUB, SUB)], ibufs[b], isems[b]
        )

    pltpu.sync_copy(part_hbm, pbuf)
    pltpu.sync_copy(p_hbm, sbuf)
    tot_f = pbuf[...].astype(jnp.float32)
    p_vec = sbuf[...]
    a_vec = p_vec / tot_f
    b_vec = (jnp.float32(1.0) - p_vec) / tot_f
    zero = jnp.zeros((L,), jnp.float32)

    def group(g, carry):
        for b in range(2):
            pltpu.make_async_copy(
                seq_hbm.at[pl.ds(base, SUB)], ibufs[b], isems[b]
            ).wait()

            @pl.when(g >= 1)
            def _(b=b):
                off0 = base
                pltpu.make_async_copy(
                    r0bufs[b], out_hbm.at[0, pl.ds(off0, SUB)], osems[0][b]
                ).wait()
                pltpu.make_async_copy(
                    r1bufs[b], out_hbm.at[1, pl.ds(off0, SUB)], osems[1][b]
                ).wait()

            @plsc.parallel_loop(0, SUB // L, step=4, unroll=4)
            def _(i, ibuf=ibufs[b], r0=r0bufs[b], r1=r1bufs[b]):
                for u in range(4):
                    off = (i + u) * L
                    x = ibuf[pl.ds(off, L)]
                    m = x == 1
                    r0[pl.ds(off, L)] = jnp.where(m, a_vec, zero)
                    r1[pl.ds(off, L)] = jnp.where(m, b_vec, zero)

            off = base + (2 * g + b) * SUB
            pltpu.async_copy(
                r0bufs[b], out_hbm.at[0, pl.ds(off, SUB)], osems[0][b]
            )
            pltpu.async_copy(
                r1bufs[b], out_hbm.at[1, pl.ds(off, SUB)], osems[1][b]
            )

            @pl.when(g + 1 < NG)
            def _(b=b):
                pltpu.async_copy(
                    seq_hbm.at[pl.ds(base + ((g + 1) * 2 + b) * SUB, SUB)],
                    ibufs[b],
                    isems[b],
                )

        return carry

    lax.fori_loop(0, NG, group, jnp.int32(0))
    for b in range(2):
        pltpu.make_async_copy(
            r0bufs[b], out_hbm.at[0, pl.ds(base, SUB)], osems[0][b]
        ).wait()
        pltpu.make_async_copy(
            r1bufs[b], out_hbm.at[1, pl.ds(base, SUB)], osems[1][b]
        ).wait()


_params = pltpu.CompilerParams(needs_layout_passes=False)

_count_call = pl.pallas_call(
    _count_tc_body,
    grid=(_TCGRID,),
    in_specs=[pl.BlockSpec((_TCROWS, 128), lambda i: (i, 0))],
    out_specs=pl.BlockSpec((L,), lambda i: (0,)),
    out_shape=jax.ShapeDtypeStruct((L,), jnp.int32),
    scratch_shapes=[pltpu.VMEM((_TCROWS, 128), jnp.int32)],
)

_scale_call = pl.kernel(
    _scale_body,
    out_type=jax.ShapeDtypeStruct((2, N), jnp.float32),
    mesh=_mesh,
    compiler_params=_params,
    scratch_types=[
        pltpu.VMEM((L,), jnp.int32),
        pltpu.VMEM((L,), jnp.float32),
        pltpu.VMEM((SUB,), jnp.int32),
        pltpu.VMEM((SUB,), jnp.int32),
        pltpu.VMEM((SUB,), jnp.float32),
        pltpu.VMEM((SUB,), jnp.float32),
        pltpu.VMEM((SUB,), jnp.float32),
        pltpu.VMEM((SUB,), jnp.float32),
        pltpu.SemaphoreType.DMA,
        pltpu.SemaphoreType.DMA,
        pltpu.SemaphoreType.DMA,
        pltpu.SemaphoreType.DMA,
        pltpu.SemaphoreType.DMA,
        pltpu.SemaphoreType.DMA,
    ],
)


def kernel(sequence, replication_prob, ber_short_patch_prob, ber_long_patch_prob):
    seq = sequence.astype(jnp.int32)
    p16 = jnp.broadcast_to(replication_prob.astype(jnp.float32), (L,))
    partials = _count_call(seq.reshape(N // 128, 128))
    return _scale_call(seq, partials, p16)
